# baseline (speedup 1.0000x reference)
import functools

import jax
import jax.numpy as jnp
from jax import lax
from jax.experimental import pallas as pl
from jax.experimental.pallas import tpu as pltpu
from jax.experimental.pallas import tpu_sc as plsc

B = 16384
SMILES_EMB = 64
CELL_EMB = 32
TARGET_DIM = 978
SMILES_VOCAB_ROWS = 1000000
CELL_VOCAB_ROWS = 1000

_NC = 2
_NS = 16
_NW = _NC * _NS
_BPW = B // _NW
_CHUNK = 256


def _sc_gather(smiles, cell, ptab_i, ctab_i):
    mesh = plsc.VectorSubcoreMesh(
        core_axis_name="c", subcore_axis_name="s", num_cores=_NC, num_subcores=_NS
    )

    @functools.partial(
        pl.kernel,
        out_type=(
            jax.ShapeDtypeStruct((B, SMILES_EMB // 2), jnp.int32),
            jax.ShapeDtypeStruct((B, CELL_EMB // 2), jnp.int32),
        ),
        mesh=mesh,
        scratch_types=[
            pltpu.VMEM((_CHUNK,), jnp.int32),
            pltpu.VMEM((_CHUNK, SMILES_EMB // 2), jnp.int32),
            pltpu.VMEM((_CHUNK, CELL_EMB // 2), jnp.int32),
            pltpu.SemaphoreType.DMA,
            pltpu.SemaphoreType.DMA,
        ],
        compiler_params=pltpu.CompilerParams(use_tc_tiling_on_sc=True),
    )
    def gather_kernel(
        smiles_hbm, cell_hbm, ptab_hbm, ctab_hbm,
        pout_hbm, cout_hbm,
        idx_v, prow_v, crow_v, psem, csem,
    ):
        wid = lax.axis_index("s") * _NC + lax.axis_index("c")
        base = wid * _BPW

        for c in range(_BPW // _CHUNK):
            cbase = base + c * _CHUNK
            pltpu.sync_copy(smiles_hbm.at[pl.ds(cbase, _CHUNK)], idx_v)

            def pbody(g, carry):
                svec = idx_v[pl.ds(g * 16, 16)]
                for j in range(16):
                    sidx = svec[j]
                    pltpu.make_async_copy(
                        ptab_hbm.at[sidx], prow_v.at[g * 16 + j], psem
                    ).start()
                return carry

            lax.fori_loop(0, _CHUNK // 16, pbody, 0)
            pltpu.make_async_copy(ptab_hbm.at[pl.ds(0, _CHUNK)], prow_v, psem).wait()
            pltpu.sync_copy(prow_v, pout_hbm.at[pl.ds(cbase, _CHUNK)])

            pltpu.sync_copy(cell_hbm.at[pl.ds(cbase, _CHUNK)], idx_v)

            def cbody(g, carry):
                cvec = idx_v[pl.ds(g * 16, 16)]
                for j in range(16):
                    cidx = cvec[j]
                    pltpu.make_async_copy(
                        ctab_hbm.at[cidx], crow_v.at[g * 16 + j], csem
                    ).start()
                return carry

            lax.fori_loop(0, _CHUNK // 16, cbody, 0)
            pltpu.make_async_copy(ctab_hbm.at[pl.ds(0, _CHUNK)], crow_v, csem).wait()
            pltpu.sync_copy(crow_v, cout_hbm.at[pl.ds(cbase, _CHUNK)])

    return gather_kernel(smiles, cell, ptab_i, ctab_i)


_TB = 1024


def _mm_body(pre_ref, pemb_ref, cemb_ref, wt_ref, wp_ref, wc_ref, b_ref, out_ref):
    dnt = (((1,), (0,)), ((), ()))
    pre_bf = pre_ref[...].astype(jnp.bfloat16)
    pemb_bf = pemb_ref[...].astype(jnp.bfloat16)
    cemb_bf = cemb_ref[...].astype(jnp.bfloat16)
    acc = lax.dot_general(wt_ref[...], pre_bf, dnt,
                          preferred_element_type=jnp.float32)
    acc += lax.dot_general(wp_ref[...], pemb_bf, dnt,
                           preferred_element_type=jnp.float32)
    acc += lax.dot_general(wc_ref[...], cemb_bf, dnt,
                           preferred_element_type=jnp.float32)
    out_ref[...] = acc + b_ref[...]


def _tc_matmul_t(pre_t, pemb_t, cemb_t, wt, wp, wc, b2d):
    grid = (B // _TB,)
    return pl.pallas_call(
        _mm_body,
        grid=grid,
        in_specs=[
            pl.BlockSpec((TARGET_DIM, _TB), lambda i: (0, i)),
            pl.BlockSpec((SMILES_EMB, _TB), lambda i: (0, i)),
            pl.BlockSpec((CELL_EMB, _TB), lambda i: (0, i)),
            pl.BlockSpec((TARGET_DIM, TARGET_DIM), lambda i: (0, 0)),
            pl.BlockSpec((TARGET_DIM, SMILES_EMB), lambda i: (0, 0)),
            pl.BlockSpec((TARGET_DIM, CELL_EMB), lambda i: (0, 0)),
            pl.BlockSpec((TARGET_DIM, 1), lambda i: (0, 0)),
        ],
        out_specs=pl.BlockSpec((TARGET_DIM, _TB), lambda i: (0, i)),
        out_shape=jax.ShapeDtypeStruct((TARGET_DIM, B), jnp.float32),
    )(pre_t, pemb_t, cemb_t, wt, wp, wc, b2d)


def kernel(smiles, cell, pre_treatment, pert_table, cell_table, W, b):
    wp = W[:, :SMILES_EMB]
    wc = W[:, SMILES_EMB:SMILES_EMB + CELL_EMB]
    wt = W[:, SMILES_EMB + CELL_EMB:]
    ptab_i = lax.bitcast_convert_type(
        pert_table.astype(jnp.bfloat16).reshape(SMILES_VOCAB_ROWS, SMILES_EMB // 2, 2),
        jnp.int32)
    ctab_i = lax.bitcast_convert_type(
        cell_table.astype(jnp.bfloat16).reshape(CELL_VOCAB_ROWS, CELL_EMB // 2, 2),
        jnp.int32)
    pemb_i, cemb_i = _sc_gather(smiles, cell, ptab_i, ctab_i)
    pemb = lax.bitcast_convert_type(pemb_i, jnp.bfloat16).reshape(B, SMILES_EMB)
    cemb = lax.bitcast_convert_type(cemb_i, jnp.bfloat16).reshape(B, CELL_EMB)
    out_t = _tc_matmul_t(
        pre_treatment.T, pemb.T, cemb.T,
        wt.astype(jnp.bfloat16), wp.astype(jnp.bfloat16),
        wc.astype(jnp.bfloat16), b.reshape(-1, 1)
    )
    return out_t.T


# speedup vs baseline: 3.6204x; 3.6204x over previous
import functools

import jax
import jax.numpy as jnp
from jax import lax
from jax.experimental import pallas as pl
from jax.experimental.pallas import tpu as pltpu
from jax.experimental.pallas import tpu_sc as plsc

B = 16384
SMILES_EMB = 64
CELL_EMB = 32
TARGET_DIM = 978
SMILES_VOCAB_ROWS = 1000000
CELL_VOCAB_ROWS = 1000

_NC = 2
_NS = 16
_NW = _NC * _NS
_BPW = B // _NW
_CHUNK = 256


def _sc_gather(smiles, cell, pert_table, cell_table):
    mesh = plsc.VectorSubcoreMesh(
        core_axis_name="c", subcore_axis_name="s", num_cores=_NC, num_subcores=_NS
    )

    @functools.partial(
        pl.kernel,
        out_type=(
            jax.ShapeDtypeStruct((B, SMILES_EMB), jnp.float32),
            jax.ShapeDtypeStruct((B, CELL_EMB), jnp.float32),
        ),
        mesh=mesh,
        scratch_types=[
            pltpu.VMEM((_CHUNK,), jnp.int32),
            pltpu.VMEM((_CHUNK, SMILES_EMB), jnp.float32),
            pltpu.VMEM((_CHUNK, CELL_EMB), jnp.float32),
            pltpu.SemaphoreType.DMA,
            pltpu.SemaphoreType.DMA,
        ],
        compiler_params=pltpu.CompilerParams(use_tc_tiling_on_sc=True),
    )
    def gather_kernel(
        smiles_hbm, cell_hbm, ptab_hbm, ctab_hbm,
        pout_hbm, cout_hbm,
        idx_v, prow_v, crow_v, psem, csem,
    ):
        wid = lax.axis_index("s") * _NC + lax.axis_index("c")
        base = wid * _BPW

        for c in range(_BPW // _CHUNK):
            cbase = base + c * _CHUNK
            pltpu.sync_copy(smiles_hbm.at[pl.ds(cbase, _CHUNK)], idx_v)

            def pbody(g, carry):
                svec = idx_v[pl.ds(g * 16, 16)]
                for j in range(16):
                    sidx = svec[j]
                    pltpu.make_async_copy(
                        ptab_hbm.at[sidx], prow_v.at[g * 16 + j], psem
                    ).start()
                return carry

            lax.fori_loop(0, _CHUNK // 16, pbody, 0)
            pltpu.make_async_copy(ptab_hbm.at[pl.ds(0, _CHUNK)], prow_v, psem).wait()
            pltpu.sync_copy(prow_v, pout_hbm.at[pl.ds(cbase, _CHUNK)])

            pltpu.sync_copy(cell_hbm.at[pl.ds(cbase, _CHUNK)], idx_v)

            def cbody(g, carry):
                cvec = idx_v[pl.ds(g * 16, 16)]
                for j in range(16):
                    cidx = cvec[j]
                    pltpu.make_async_copy(
                        ctab_hbm.at[cidx], crow_v.at[g * 16 + j], csem
                    ).start()
                return carry

            lax.fori_loop(0, _CHUNK // 16, cbody, 0)
            pltpu.make_async_copy(ctab_hbm.at[pl.ds(0, _CHUNK)], crow_v, csem).wait()
            pltpu.sync_copy(crow_v, cout_hbm.at[pl.ds(cbase, _CHUNK)])

    return gather_kernel(smiles, cell, pert_table, cell_table)


_TB = 1024


def _mm_body(pre_ref, pemb_ref, cemb_ref, wt_ref, wp_ref, wc_ref, b_ref, out_ref):
    dnt = (((1,), (0,)), ((), ()))
    acc = lax.dot_general(wt_ref[...], pre_ref[...], dnt,
                          preferred_element_type=jnp.float32)
    acc += lax.dot_general(wp_ref[...], pemb_ref[...], dnt,
                           preferred_element_type=jnp.float32)
    acc += lax.dot_general(wc_ref[...], cemb_ref[...], dnt,
                           preferred_element_type=jnp.float32)
    out_ref[...] = acc + b_ref[...]


def _tc_matmul_t(pre_t, pemb_t, cemb_t, wt, wp, wc, b2d):
    grid = (B // _TB,)
    return pl.pallas_call(
        _mm_body,
        grid=grid,
        in_specs=[
            pl.BlockSpec((TARGET_DIM, _TB), lambda i: (0, i)),
            pl.BlockSpec((SMILES_EMB, _TB), lambda i: (0, i)),
            pl.BlockSpec((CELL_EMB, _TB), lambda i: (0, i)),
            pl.BlockSpec((TARGET_DIM, TARGET_DIM), lambda i: (0, 0)),
            pl.BlockSpec((TARGET_DIM, SMILES_EMB), lambda i: (0, 0)),
            pl.BlockSpec((TARGET_DIM, CELL_EMB), lambda i: (0, 0)),
            pl.BlockSpec((TARGET_DIM, 1), lambda i: (0, 0)),
        ],
        out_specs=pl.BlockSpec((TARGET_DIM, _TB), lambda i: (0, i)),
        out_shape=jax.ShapeDtypeStruct((TARGET_DIM, B), jnp.float32),
    )(pre_t, pemb_t, cemb_t, wt, wp, wc, b2d)


def kernel(smiles, cell, pre_treatment, pert_table, cell_table, W, b):
    wp = W[:, :SMILES_EMB]
    wc = W[:, SMILES_EMB:SMILES_EMB + CELL_EMB]
    wt = W[:, SMILES_EMB + CELL_EMB:]
    pemb, cemb = _sc_gather(smiles, cell, pert_table, cell_table)
    out_t = _tc_matmul_t(
        pre_treatment.T, pemb.T, cemb.T, wt, wp, wc, b.reshape(-1, 1)
    )
    return out_t.T


# speedup vs baseline: 3.6367x; 1.0045x over previous
import functools

import jax
import jax.numpy as jnp
from jax import lax
from jax.experimental import pallas as pl
from jax.experimental.pallas import tpu as pltpu
from jax.experimental.pallas import tpu_sc as plsc

B = 16384
SMILES_EMB = 64
CELL_EMB = 32
TARGET_DIM = 978
SMILES_VOCAB_ROWS = 1000000
CELL_VOCAB_ROWS = 1000

_NC = 2
_NS = 16
_NW = _NC * _NS
_BPW = B // _NW
_CHUNK = 256


def _sc_gather(smiles, cell, pert_table, cell_table):
    mesh = plsc.VectorSubcoreMesh(
        core_axis_name="c", subcore_axis_name="s", num_cores=_NC, num_subcores=_NS
    )

    @functools.partial(
        pl.kernel,
        out_type=(
            jax.ShapeDtypeStruct((B, SMILES_EMB), jnp.float32),
            jax.ShapeDtypeStruct((B, CELL_EMB), jnp.float32),
        ),
        mesh=mesh,
        scratch_types=[
            pltpu.VMEM((_CHUNK,), jnp.int32),
            pltpu.VMEM((_CHUNK, SMILES_EMB), jnp.float32),
            pltpu.VMEM((_CHUNK, CELL_EMB), jnp.float32),
            pltpu.SemaphoreType.DMA,
            pltpu.SemaphoreType.DMA,
        ],
        compiler_params=pltpu.CompilerParams(use_tc_tiling_on_sc=True),
    )
    def gather_kernel(
        smiles_hbm, cell_hbm, ptab_hbm, ctab_hbm,
        pout_hbm, cout_hbm,
        idx_v, prow_v, crow_v, psem, csem,
    ):
        wid = lax.axis_index("s") * _NC + lax.axis_index("c")
        base = wid * _BPW

        for c in range(_BPW // _CHUNK):
            cbase = base + c * _CHUNK
            pltpu.sync_copy(smiles_hbm.at[pl.ds(cbase, _CHUNK)], idx_v)

            def pbody(g, carry):
                svec = idx_v[pl.ds(g * 16, 16)]
                for j in range(16):
                    sidx = svec[j]
                    pltpu.make_async_copy(
                        ptab_hbm.at[sidx], prow_v.at[g * 16 + j], psem
                    ).start()
                return carry

            lax.fori_loop(0, _CHUNK // 16, pbody, 0)
            pltpu.make_async_copy(ptab_hbm.at[pl.ds(0, _CHUNK)], prow_v, psem).wait()
            pltpu.sync_copy(prow_v, pout_hbm.at[pl.ds(cbase, _CHUNK)])

            pltpu.sync_copy(cell_hbm.at[pl.ds(cbase, _CHUNK)], idx_v)

            def cbody(g, carry):
                cvec = idx_v[pl.ds(g * 16, 16)]
                for j in range(16):
                    cidx = cvec[j]
                    pltpu.make_async_copy(
                        ctab_hbm.at[cidx], crow_v.at[g * 16 + j], csem
                    ).start()
                return carry

            lax.fori_loop(0, _CHUNK // 16, cbody, 0)
            pltpu.make_async_copy(ctab_hbm.at[pl.ds(0, _CHUNK)], crow_v, csem).wait()
            pltpu.sync_copy(crow_v, cout_hbm.at[pl.ds(cbase, _CHUNK)])

    return gather_kernel(smiles, cell, pert_table, cell_table)


_TB = 2048


def _mm_body(pre_ref, pemb_ref, cemb_ref, wt_ref, wp_ref, wc_ref, b_ref, out_ref):
    dnt = (((1,), (0,)), ((), ()))
    acc = lax.dot_general(wt_ref[...], pre_ref[...], dnt,
                          preferred_element_type=jnp.float32)
    acc += lax.dot_general(wp_ref[...], pemb_ref[...], dnt,
                           preferred_element_type=jnp.float32)
    acc += lax.dot_general(wc_ref[...], cemb_ref[...], dnt,
                           preferred_element_type=jnp.float32)
    out_ref[...] = acc + b_ref[...]


def _tc_matmul_t(pre_t, pemb_t, cemb_t, wt, wp, wc, b2d):
    grid = (B // _TB,)
    return pl.pallas_call(
        _mm_body,
        grid=grid,
        in_specs=[
            pl.BlockSpec((TARGET_DIM, _TB), lambda i: (0, i)),
            pl.BlockSpec((SMILES_EMB, _TB), lambda i: (0, i)),
            pl.BlockSpec((CELL_EMB, _TB), lambda i: (0, i)),
            pl.BlockSpec((TARGET_DIM, TARGET_DIM), lambda i: (0, 0)),
            pl.BlockSpec((TARGET_DIM, SMILES_EMB), lambda i: (0, 0)),
            pl.BlockSpec((TARGET_DIM, CELL_EMB), lambda i: (0, 0)),
            pl.BlockSpec((TARGET_DIM, 1), lambda i: (0, 0)),
        ],
        out_specs=pl.BlockSpec((TARGET_DIM, _TB), lambda i: (0, i)),
        out_shape=jax.ShapeDtypeStruct((TARGET_DIM, B), jnp.float32),
    )(pre_t, pemb_t, cemb_t, wt, wp, wc, b2d)


def kernel(smiles, cell, pre_treatment, pert_table, cell_table, W, b):
    wp = W[:, :SMILES_EMB]
    wc = W[:, SMILES_EMB:SMILES_EMB + CELL_EMB]
    wt = W[:, SMILES_EMB + CELL_EMB:]
    pemb, cemb = _sc_gather(smiles, cell, pert_table, cell_table)
    out_t = _tc_matmul_t(
        pre_treatment.T, pemb.T, cemb.T, wt, wp, wc, b.reshape(-1, 1)
    )
    return out_t.T


# speedup vs baseline: 3.6601x; 1.0064x over previous
import functools

import jax
import jax.numpy as jnp
from jax import lax
from jax.experimental import pallas as pl
from jax.experimental.pallas import tpu as pltpu
from jax.experimental.pallas import tpu_sc as plsc

B = 16384
SMILES_EMB = 64
CELL_EMB = 32
TARGET_DIM = 978

_NC = 2
_NS = 16
_NW = _NC * _NS
_BPW = B // _NW
_CHUNK = 256


def _sc_gather(smiles, cell, pert_table, cell_table):
    mesh = plsc.VectorSubcoreMesh(
        core_axis_name="c", subcore_axis_name="s", num_cores=_NC, num_subcores=_NS
    )

    @functools.partial(
        pl.kernel,
        out_type=(
            jax.ShapeDtypeStruct((B, SMILES_EMB), jnp.float32),
            jax.ShapeDtypeStruct((B, CELL_EMB), jnp.float32),
        ),
        mesh=mesh,
        scratch_types=[
            pltpu.VMEM((_CHUNK,), jnp.int32),
            pltpu.VMEM((_CHUNK, SMILES_EMB), jnp.float32),
            pltpu.VMEM((_CHUNK, CELL_EMB), jnp.float32),
            pltpu.SemaphoreType.DMA,
            pltpu.SemaphoreType.DMA,
        ],
        compiler_params=pltpu.CompilerParams(use_tc_tiling_on_sc=True),
    )
    def gather_kernel(
        smiles_hbm, cell_hbm, ptab_hbm, ctab_hbm,
        pout_hbm, cout_hbm,
        idx_v, prow_v, crow_v, psem, csem,
    ):
        wid = lax.axis_index("s") * _NC + lax.axis_index("c")
        base = wid * _BPW

        for c in range(_BPW // _CHUNK):
            cbase = base + c * _CHUNK
            pltpu.sync_copy(smiles_hbm.at[pl.ds(cbase, _CHUNK)], idx_v)

            def pbody(g, carry):
                svec = idx_v[pl.ds(g * 16, 16)]
                for j in range(16):
                    sidx = svec[j]
                    pltpu.make_async_copy(
                        ptab_hbm.at[sidx], prow_v.at[g * 16 + j], psem
                    ).start()
                return carry

            lax.fori_loop(0, _CHUNK // 16, pbody, 0)
            pltpu.make_async_copy(ptab_hbm.at[pl.ds(0, _CHUNK)], prow_v, psem).wait()
            pltpu.sync_copy(prow_v, pout_hbm.at[pl.ds(cbase, _CHUNK)])

            pltpu.sync_copy(cell_hbm.at[pl.ds(cbase, _CHUNK)], idx_v)

            def cbody(g, carry):
                cvec = idx_v[pl.ds(g * 16, 16)]
                for j in range(16):
                    cidx = cvec[j]
                    pltpu.make_async_copy(
                        ctab_hbm.at[cidx], crow_v.at[g * 16 + j], csem
                    ).start()
                return carry

            lax.fori_loop(0, _CHUNK // 16, cbody, 0)
            pltpu.make_async_copy(ctab_hbm.at[pl.ds(0, _CHUNK)], crow_v, csem).wait()
            pltpu.sync_copy(crow_v, cout_hbm.at[pl.ds(cbase, _CHUNK)])

    return gather_kernel(smiles, cell, pert_table, cell_table)


_TB = 2048


def _mm_body(pre_ref, pemb_ref, cemb_ref, wt_ref, wp_ref, wc_ref, b_ref, out_ref):
    dnt = (((1,), (0,)), ((), ()))
    acc = lax.dot_general(wt_ref[...], pre_ref[...], dnt,
                          preferred_element_type=jnp.float32)
    acc += lax.dot_general(wp_ref[...], pemb_ref[...], dnt,
                           preferred_element_type=jnp.float32)
    acc += lax.dot_general(wc_ref[...], cemb_ref[...], dnt,
                           preferred_element_type=jnp.float32)
    out_ref[...] = acc + b_ref[...]


def _tc_matmul_t(pre_t, pemb_t, cemb_t, wt, wp, wc, b2d):
    grid = (B // _TB,)
    return pl.pallas_call(
        _mm_body,
        grid=grid,
        in_specs=[
            pl.BlockSpec((TARGET_DIM, _TB), lambda i: (0, i)),
            pl.BlockSpec((SMILES_EMB, _TB), lambda i: (0, i)),
            pl.BlockSpec((CELL_EMB, _TB), lambda i: (0, i)),
            pl.BlockSpec((TARGET_DIM, TARGET_DIM), lambda i: (0, 0)),
            pl.BlockSpec((TARGET_DIM, SMILES_EMB), lambda i: (0, 0)),
            pl.BlockSpec((TARGET_DIM, CELL_EMB), lambda i: (0, 0)),
            pl.BlockSpec((TARGET_DIM, 1), lambda i: (0, 0)),
        ],
        out_specs=pl.BlockSpec((TARGET_DIM, _TB), lambda i: (0, i)),
        out_shape=jax.ShapeDtypeStruct((TARGET_DIM, B), jnp.float32),
    )(pre_t, pemb_t, cemb_t, wt, wp, wc, b2d)


def kernel(smiles, cell, pre_treatment, pert_table, cell_table, W, b):
    wp = W[:, :SMILES_EMB]
    wc = W[:, SMILES_EMB:SMILES_EMB + CELL_EMB]
    wt = W[:, SMILES_EMB + CELL_EMB:]
    pemb, cemb = _sc_gather(smiles, cell, pert_table, cell_table)
    out_t = _tc_matmul_t(
        pre_treatment.T, pemb.T, cemb.T, wt, wp, wc, b.reshape(-1, 1)
    )
    return out_t.T
